# baseline (speedup 1.0000x reference)
import functools

import jax
import jax.numpy as jnp
from jax import lax
from jax.experimental import pallas as pl
from jax.experimental.pallas import tpu as pltpu
from jax.experimental.pallas import tpu_sc as plsc

NC = 2
NS = 16
NW = NC * NS
DW = 128
PW = 16
NP = 10240
CH = 80
CHP = 1000
CHS = 200
CHT = 1000


def _silu(v):
    return v * (0.5 * jnp.tanh(0.5 * v) + 0.5)


def _gather_body(epw, ch, t_hbm, row_hbm, col_hbm, outr_hbm, outc_hbm,
                 idxr, idxc, bufr, bufc, sem):
    wid = lax.axis_index("s") * NC + lax.axis_index("c")
    base0 = wid * epw

    def step(i, carry):
        base = base0 + i * ch
        pltpu.sync_copy(row_hbm.at[pl.ds(base, ch)], idxr)
        pltpu.sync_copy(col_hbm.at[pl.ds(base, ch)], idxc)
        a = pltpu.async_copy(t_hbm.at[idxr], bufr, sem)
        b = pltpu.async_copy(t_hbm.at[idxc], bufc, sem)
        a.wait()
        b.wait()
        pltpu.sync_copy(bufr, outr_hbm.at[pl.ds(base, ch)])
        pltpu.sync_copy(bufc, outc_hbm.at[pl.ds(base, ch)])
        return carry

    lax.fori_loop(0, epw // ch, step, 0)


def _gather_x_body(epw, n, t_hbm, row_hbm, col_hbm, outr_hbm, outc_hbm,
                   idxr, idxc, bufr, bufc, tab, sem):
    cid = lax.axis_index("c")
    sid = lax.axis_index("s")
    wid = sid * NC + cid
    @pl.when(sid == 0)
    def _():
        pltpu.sync_copy(t_hbm, tab)

    plsc.subcore_barrier()
    base0 = wid * epw

    def step(i, carry):
        base = base0 + i * CH
        pltpu.sync_copy(row_hbm.at[pl.ds(base, CH)], idxr)
        pltpu.sync_copy(col_hbm.at[pl.ds(base, CH)], idxc)
        a = pltpu.async_copy(tab.at[idxr], bufr, sem)
        b = pltpu.async_copy(tab.at[idxc], bufc, sem)
        a.wait()
        b.wait()
        pltpu.sync_copy(bufr, outr_hbm.at[pl.ds(base, CH)])
        pltpu.sync_copy(bufc, outc_hbm.at[pl.ds(base, CH)])
        return carry

    lax.fori_loop(0, epw // CH, step, 0)


def _gather_x(table, row, col, E):
    n = table.shape[0]
    epw = E // NW
    return pl.kernel(
        functools.partial(_gather_x_body, epw, n),
        out_type=[jax.ShapeDtypeStruct((E, DW), jnp.float32),
                  jax.ShapeDtypeStruct((E, DW), jnp.float32)],
        mesh=plsc.VectorSubcoreMesh(core_axis_name="c", subcore_axis_name="s"),
        scratch_types=[
            pltpu.VMEM((CH,), jnp.int32),
            pltpu.VMEM((CH,), jnp.int32),
            pltpu.VMEM((CH, DW), jnp.float32),
            pltpu.VMEM((CH, DW), jnp.float32),
            pltpu.VMEM_SHARED((10000, DW), jnp.float32),
            pltpu.SemaphoreType.DMA,
        ],
    )(table, row, col)


def _gather_pos(table, row, col, E):
    epw = E // NW
    return pl.kernel(
        functools.partial(_gather_body, epw, CHP),
        out_type=[jax.ShapeDtypeStruct((E, PW), jnp.float32),
                  jax.ShapeDtypeStruct((E, PW), jnp.float32)],
        mesh=plsc.VectorSubcoreMesh(core_axis_name="c", subcore_axis_name="s"),
        scratch_types=[
            pltpu.VMEM((CHP,), jnp.int32),
            pltpu.VMEM((CHP,), jnp.int32),
            pltpu.VMEM((CHP, PW), jnp.float32),
            pltpu.VMEM((CHP, PW), jnp.float32),
            pltpu.SemaphoreType.DMA,
        ],
        compiler_params=pltpu.CompilerParams(use_tc_tiling_on_sc=False),
    )(table, row, col)


def _scatter_body(epw, ch, w, payload_hbm, col_hbm, zeros_hbm, out_hbm,
                  idx, buf, acc):
    cid = lax.axis_index("c")
    sid = lax.axis_index("s")
    wid = sid * NC + cid
    rpt = NP // NS
    pltpu.sync_copy(zeros_hbm, acc.at[pl.ds(sid * rpt, rpt)])
    plsc.subcore_barrier()
    base0 = wid * epw

    def step(i, carry):
        base = base0 + i * ch
        pltpu.sync_copy(col_hbm.at[pl.ds(base, ch)], idx)
        pltpu.sync_copy(payload_hbm.at[pl.ds(base, ch)], buf)
        pltpu.sync_copy(buf, acc.at[idx], add=True)
        return carry

    lax.fori_loop(0, epw // ch, step, 0)
    plsc.subcore_barrier()
    pltpu.sync_copy(acc.at[pl.ds(sid * rpt, rpt)],
                    out_hbm.at[cid, pl.ds(sid * rpt, rpt)])


def _scatter_msg(payload, col, E):
    epw = E // NW
    zeros = jnp.zeros((NP // NS, DW), jnp.float32)
    return pl.kernel(
        functools.partial(_scatter_body, epw, CHS, DW),
        out_type=jax.ShapeDtypeStruct((NC, NP, DW), jnp.float32),
        mesh=plsc.VectorSubcoreMesh(core_axis_name="c", subcore_axis_name="s"),
        scratch_types=[
            pltpu.VMEM((CHS,), jnp.int32),
            pltpu.VMEM((CHS, DW), jnp.float32),
            pltpu.VMEM_SHARED((NP, DW), jnp.float32),
        ],
    )(payload, col, zeros)


def _scatter_tail(payload, col, E):
    epw = E // NW
    zeros = jnp.zeros((NP // NS, PW), jnp.float32)
    return pl.kernel(
        functools.partial(_scatter_body, epw, CHT, PW),
        out_type=jax.ShapeDtypeStruct((NC, NP, PW), jnp.float32),
        mesh=plsc.VectorSubcoreMesh(core_axis_name="c", subcore_axis_name="s"),
        scratch_types=[
            pltpu.VMEM((CHT,), jnp.int32),
            pltpu.VMEM((CHT, PW), jnp.float32),
            pltpu.VMEM_SHARED((NP, PW), jnp.float32),
        ],
        compiler_params=pltpu.CompilerParams(use_tc_tiling_on_sc=False),
    )(payload, col, zeros)


def _shl(x, k):
    z = jnp.zeros((x.shape[0], k), jnp.float32)
    return jnp.concatenate([x[:, k:], z], axis=1)


def _shr(x, k):
    z = jnp.zeros((x.shape[0], k), jnp.float32)
    return jnp.concatenate([z, x[:, :-k]], axis=1)


def _edge_body(D, xr_ref, xc_ref, pr_ref, pc_ref, rbf_ref,
               wr_ref, wc_ref, wb_ref, b1_ref,
               mW2_ref, mb2_ref, mW3_ref, mb3_ref,
               cW2_ref, cb2_ref, cW3c_ref, cb3_ref,
               aW2c_ref, ab2_ref, sw_ref, sc_ref, msg_ref, tail_ref):
    f32 = jnp.float32
    bf16 = jnp.bfloat16
    xr = xr_ref[...].astype(bf16)
    xc = xc_ref[...].astype(bf16)
    pre1 = (jnp.dot(xr, wr_ref[...], preferred_element_type=f32)
            + jnp.dot(xc, wc_ref[...], preferred_element_type=f32)
            + lax.dot_general(rbf_ref[...].astype(bf16), wb_ref[...],
                              (((0,), (0,)), ((), ())),
                              preferred_element_type=f32)
            + b1_ref[...])
    hm = _silu(pre1[:, :D])
    hc = _silu(pre1[:, D:2 * D])
    ha = _silu(pre1[:, 2 * D:])
    h2 = _silu(jnp.dot(hm.astype(bf16), mW2_ref[...],
                       preferred_element_type=f32) + mb2_ref[...])
    msg = jnp.dot(h2.astype(bf16), mW3_ref[...],
                  preferred_element_type=f32) + mb3_ref[...]
    c2 = _silu(jnp.dot(hc.astype(bf16), cW2_ref[...],
                       preferred_element_type=f32) + cb2_ref[...])
    cw = jnp.dot(c2, cW3c_ref[...], preferred_element_type=f32) + cb3_ref[...]
    logit = (jnp.dot(ha, aW2c_ref[...], preferred_element_type=f32)
             + ab2_ref[...])
    w = jnp.exp(logit)
    msg_ref[...] = msg * w

    blk8 = msg.shape[0] // 8
    vec = pc_ref[...] - pr_ref[...]
    sq = vec * vec
    d2 = sq + _shl(sq, 1) + _shl(sq, 2)
    inv = 1.0 / (jnp.sqrt(d2 + 1e-12) + 1e-8)
    lane = lax.broadcasted_iota(jnp.int32, (blk8, 128), 1)
    m0 = (lane & 15) == 0
    invm = jnp.where(m0, inv, 0.0)
    invb = _shr(invm, 1) + _shr(invm, 2) + _shr(invm, 3)
    dirp = _shr(vec, 1) * invb
    l8 = jnp.reshape(logit, (blk8, 8))
    c8 = jnp.reshape(cw, (blk8, 8))
    lp = jnp.dot(l8, sw_ref[...], preferred_element_type=f32)
    cp = jnp.dot(c8, sc_ref[...], preferred_element_type=f32)
    wp = jnp.where(m0, jnp.exp(lp), 0.0)
    tail_ref[...] = wp + cp * dirp


def _edge_mlps(xr, xc, pr8, pc8, rbf, Wr, Wc, Wb, b1, mW2, mb2r, mW3, mb3r,
               cW2, cb2r, cW3c, cb3r, aW2c, ab2r, Sw, Sc, E, D, NB, W1):
    BLK = 3200
    grid = (E // BLK,)
    espec = lambda wdt: pl.BlockSpec((BLK, wdt), lambda i: (i, 0))
    pspec = pl.BlockSpec((BLK // 8, 128), lambda i: (i, 0))
    full = lambda s: pl.BlockSpec(s, lambda i: (0, 0))
    return pl.pallas_call(
        functools.partial(_edge_body, D),
        grid=grid,
        in_specs=[
            espec(DW), espec(DW), pspec, pspec,
            pl.BlockSpec((NB, BLK), lambda i: (0, i)),
            full((D, W1)), full((D, W1)), full((NB, W1)), full((1, W1)),
            full((D, D)), full((1, D)), full((D, D)), full((1, D)),
            full((D, D)), full((1, D)), full((D, 1)), full((1, 1)),
            full((D // 2, 1)), full((1, 1)), full((8, 128)), full((8, 128)),
        ],
        out_specs=[espec(DW), pspec],
        out_shape=[jax.ShapeDtypeStruct((E, DW), jnp.float32),
                   jax.ShapeDtypeStruct((E // 8, 128), jnp.float32)],
        compiler_params=pltpu.CompilerParams(
            dimension_semantics=("parallel",)),
    )(xr, xc, pr8, pc8, rbf, Wr, Wc, Wb, b1, mW2, mb2r, mW3, mb3r,
      cW2, cb2r, cW3c, cb3r, aW2c, ab2r, Sw, Sc)


def _node_body(D, x_ref, pos_ref, m0_ref, m1_ref, t0_ref, t1_ref,
               uW1a_ref, uW1b_ref, ub1_ref, uW2_ref, ub2_ref,
               g_ref, b_ref, xo_ref, po_ref):
    f32 = jnp.float32
    num = m0_ref[...] + m1_ref[...]
    den = t0_ref[:, 0:1] + t1_ref[:, 0:1]
    ca = t0_ref[:, 1:4] + t1_ref[:, 1:4]
    m = num / (den + 1e-8)
    x = x_ref[...]
    h = _silu(jnp.dot(x, uW1a_ref[...], preferred_element_type=f32)
              + jnp.dot(m, uW1b_ref[...], preferred_element_type=f32)
              + ub1_ref[...])
    u = jnp.dot(h, uW2_ref[...], preferred_element_type=f32) + ub2_ref[...]
    pre = x + u
    mean = jnp.mean(pre, axis=1, keepdims=True)
    cent = pre - mean
    var = jnp.mean(cent * cent, axis=1, keepdims=True)
    xo_ref[...] = g_ref[...] * cent / jnp.sqrt(var + 1e-5) + b_ref[...]
    po_ref[...] = pos_ref[...] + ca


def _node_update(x, pos, m0, m1, t0, t1, uW1a, uW1b, ub1r, uW2, ub2r,
                 gr, br, n, D):
    BNK = 1000
    grid = (n // BNK,)
    full = lambda s: pl.BlockSpec(s, lambda i: (0, 0))
    return pl.pallas_call(
        functools.partial(_node_body, D),
        grid=grid,
        in_specs=[
            pl.BlockSpec((BNK, D), lambda i: (i, 0)),
            pl.BlockSpec((BNK, 3), lambda i: (i, 0)),
            pl.BlockSpec((BNK, DW), lambda i: (i, 0)),
            pl.BlockSpec((BNK, DW), lambda i: (i, 0)),
            pl.BlockSpec((BNK, PW), lambda i: (i, 0)),
            pl.BlockSpec((BNK, PW), lambda i: (i, 0)),
            full((D, D)), full((D, D)), full((1, D)), full((D, D)), full((1, D)),
            full((1, D)), full((1, D)),
        ],
        out_specs=[
            pl.BlockSpec((BNK, D), lambda i: (i, 0)),
            pl.BlockSpec((BNK, 3), lambda i: (i, 0)),
        ],
        out_shape=[
            jax.ShapeDtypeStruct((n, D), jnp.float32),
            jax.ShapeDtypeStruct((n, 3), jnp.float32),
        ],
        compiler_params=pltpu.CompilerParams(
            dimension_semantics=("parallel",)),
    )(x, pos, m0, m1, t0, t1, uW1a, uW1b, ub1r, uW2, ub2r, gr, br)


def kernel(x, pos, edge_index, rbf, mW1, mb1, mW2, mb2, mW3, mb3,
           aW1, ab1, aW2, ab2, cW1, cb1, cW2, cb2, cW3, cb3,
           uW1, ub1, uW2, ub2, gamma, beta):
    n, D = x.shape
    E = edge_index.shape[1]
    NB = rbf.shape[1]
    W1 = 2 * D + D // 2
    row = edge_index[0]
    col = edge_index[1]

    post = jnp.concatenate(
        [pos, jnp.zeros((n, PW - 3), jnp.float32)], axis=1)
    bf16 = jnp.bfloat16
    Wr = jnp.concatenate([mW1[:D], cW1[:D], aW1[:D]], axis=1).astype(bf16)
    Wc = jnp.concatenate([mW1[D:2 * D], cW1[D:2 * D], aW1[D:2 * D]],
                         axis=1).astype(bf16)
    Wb = jnp.concatenate([mW1[2 * D:], cW1[2 * D:], aW1[2 * D:]],
                         axis=1).astype(bf16)
    b1 = jnp.concatenate([mb1, cb1, ab1]).reshape(1, W1)

    ji = jnp.arange(8)[:, None]
    li = jnp.arange(128)[None, :]
    Sw = (li == 16 * ji).astype(jnp.float32)
    Sc = ((li >= 16 * ji + 1) & (li <= 16 * ji + 3)).astype(jnp.float32)

    xr, xc = _gather_x(x, row, col, E)
    pr, pc = _gather_pos(post, row, col, E)
    pr8 = jnp.reshape(pr, (E // 8, 128))
    pc8 = jnp.reshape(pc, (E // 8, 128))
    msg_pay, tail_pay = _edge_mlps(
        xr, xc, pr8, pc8, rbf.T, Wr, Wc, Wb, b1,
        mW2.astype(bf16), mb2.reshape(1, D), mW3.astype(bf16), mb3.reshape(1, D),
        cW2.astype(bf16), cb2.reshape(1, D), cW3.reshape(D, 1), cb3.reshape(1, 1),
        aW2.reshape(D // 2, 1), ab2.reshape(1, 1), Sw, Sc, E, D, NB, W1)
    accm = _scatter_msg(msg_pay, col, E)
    acct = _scatter_tail(jnp.reshape(tail_pay, (E, PW)), col, E)
    x_new, pos_new = _node_update(
        x, pos, accm[0, :n], accm[1, :n], acct[0, :n], acct[1, :n],
        uW1[:D], uW1[D:], ub1.reshape(1, D), uW2, ub2.reshape(1, D),
        gamma.reshape(1, D), beta.reshape(1, D), n, D)
    return (x_new, pos_new)


# speedup vs baseline: 1.0634x; 1.0634x over previous
import functools

import jax
import jax.numpy as jnp
from jax import lax
from jax.experimental import pallas as pl
from jax.experimental.pallas import tpu as pltpu
from jax.experimental.pallas import tpu_sc as plsc

NC = 2
NS = 16
NW = NC * NS
DW = 128
PW = 16
NP = 10240
CH = 400
CHP = 1000
CHS = 200
CHT = 1000


def _silu(v):
    return v * (0.5 * jnp.tanh(0.5 * v) + 0.5)


def _gather_body(epw, ch, t_hbm, row_hbm, col_hbm, outr_hbm, outc_hbm,
                 idxr, idxc, bufr, bufc, sem):
    wid = lax.axis_index("s") * NC + lax.axis_index("c")
    base0 = wid * epw

    def step(i, carry):
        base = base0 + i * ch
        pltpu.sync_copy(row_hbm.at[pl.ds(base, ch)], idxr)
        pltpu.sync_copy(col_hbm.at[pl.ds(base, ch)], idxc)
        a = pltpu.async_copy(t_hbm.at[idxr], bufr, sem)
        b = pltpu.async_copy(t_hbm.at[idxc], bufc, sem)
        a.wait()
        b.wait()
        pltpu.sync_copy(bufr, outr_hbm.at[pl.ds(base, ch)])
        pltpu.sync_copy(bufc, outc_hbm.at[pl.ds(base, ch)])
        return carry

    lax.fori_loop(0, epw // ch, step, 0)


def _gather_x(table, row, col, E):
    epw = E // NW
    return pl.kernel(
        functools.partial(_gather_body, epw, CH),
        out_type=[jax.ShapeDtypeStruct((E, DW), jnp.float32),
                  jax.ShapeDtypeStruct((E, DW), jnp.float32)],
        mesh=plsc.VectorSubcoreMesh(core_axis_name="c", subcore_axis_name="s"),
        scratch_types=[
            pltpu.VMEM((CH,), jnp.int32),
            pltpu.VMEM((CH,), jnp.int32),
            pltpu.VMEM((CH, DW), jnp.float32),
            pltpu.VMEM((CH, DW), jnp.float32),
            pltpu.SemaphoreType.DMA,
        ],
    )(table, row, col)


def _gather_pos(table, row, col, E):
    epw = E // NW
    return pl.kernel(
        functools.partial(_gather_body, epw, CHP),
        out_type=[jax.ShapeDtypeStruct((E, PW), jnp.float32),
                  jax.ShapeDtypeStruct((E, PW), jnp.float32)],
        mesh=plsc.VectorSubcoreMesh(core_axis_name="c", subcore_axis_name="s"),
        scratch_types=[
            pltpu.VMEM((CHP,), jnp.int32),
            pltpu.VMEM((CHP,), jnp.int32),
            pltpu.VMEM((CHP, PW), jnp.float32),
            pltpu.VMEM((CHP, PW), jnp.float32),
            pltpu.SemaphoreType.DMA,
        ],
        compiler_params=pltpu.CompilerParams(use_tc_tiling_on_sc=False),
    )(table, row, col)


def _scatter_body(epw, ch, w, payload_hbm, col_hbm, zeros_hbm, out_hbm,
                  idx, buf, acc):
    cid = lax.axis_index("c")
    sid = lax.axis_index("s")
    wid = sid * NC + cid
    rpt = NP // NS
    pltpu.sync_copy(zeros_hbm, acc.at[pl.ds(sid * rpt, rpt)])
    plsc.subcore_barrier()
    base0 = wid * epw

    def step(i, carry):
        base = base0 + i * ch
        pltpu.sync_copy(col_hbm.at[pl.ds(base, ch)], idx)
        pltpu.sync_copy(payload_hbm.at[pl.ds(base, ch)], buf)
        pltpu.sync_copy(buf, acc.at[idx], add=True)
        return carry

    lax.fori_loop(0, epw // ch, step, 0)
    plsc.subcore_barrier()
    pltpu.sync_copy(acc.at[pl.ds(sid * rpt, rpt)],
                    out_hbm.at[cid, pl.ds(sid * rpt, rpt)])


def _scatter_msg(payload, col, E):
    epw = E // NW
    zeros = jnp.zeros((NP // NS, DW), jnp.float32)
    return pl.kernel(
        functools.partial(_scatter_body, epw, CHS, DW),
        out_type=jax.ShapeDtypeStruct((NC, NP, DW), jnp.float32),
        mesh=plsc.VectorSubcoreMesh(core_axis_name="c", subcore_axis_name="s"),
        scratch_types=[
            pltpu.VMEM((CHS,), jnp.int32),
            pltpu.VMEM((CHS, DW), jnp.float32),
            pltpu.VMEM_SHARED((NP, DW), jnp.float32),
        ],
    )(payload, col, zeros)


def _scatter_tail(payload, col, E):
    epw = E // NW
    zeros = jnp.zeros((NP // NS, PW), jnp.float32)
    return pl.kernel(
        functools.partial(_scatter_body, epw, CHT, PW),
        out_type=jax.ShapeDtypeStruct((NC, NP, PW), jnp.float32),
        mesh=plsc.VectorSubcoreMesh(core_axis_name="c", subcore_axis_name="s"),
        scratch_types=[
            pltpu.VMEM((CHT,), jnp.int32),
            pltpu.VMEM((CHT, PW), jnp.float32),
            pltpu.VMEM_SHARED((NP, PW), jnp.float32),
        ],
        compiler_params=pltpu.CompilerParams(use_tc_tiling_on_sc=False),
    )(payload, col, zeros)


def _shl(x, k):
    z = jnp.zeros((x.shape[0], k), jnp.float32)
    return jnp.concatenate([x[:, k:], z], axis=1)


def _shr(x, k):
    z = jnp.zeros((x.shape[0], k), jnp.float32)
    return jnp.concatenate([z, x[:, :-k]], axis=1)


def _edge_body(D, xr_ref, xc_ref, pr_ref, pc_ref, rbf_ref,
               wr_ref, wc_ref, wb_ref, b1_ref,
               mW2_ref, mb2_ref, mW3_ref, mb3_ref,
               cW2_ref, cb2_ref, cW3c_ref, cb3_ref,
               aW2c_ref, ab2_ref, sw_ref, sc_ref, msg_ref, tail_ref):
    f32 = jnp.float32
    bf16 = jnp.bfloat16
    xr = xr_ref[...].astype(bf16)
    xc = xc_ref[...].astype(bf16)
    pre1 = (jnp.dot(xr, wr_ref[...], preferred_element_type=f32)
            + jnp.dot(xc, wc_ref[...], preferred_element_type=f32)
            + lax.dot_general(rbf_ref[...].astype(bf16), wb_ref[...],
                              (((0,), (0,)), ((), ())),
                              preferred_element_type=f32)
            + b1_ref[...])
    hm = _silu(pre1[:, :D])
    hc = _silu(pre1[:, D:2 * D])
    ha = _silu(pre1[:, 2 * D:])
    h2 = _silu(jnp.dot(hm.astype(bf16), mW2_ref[...],
                       preferred_element_type=f32) + mb2_ref[...])
    msg = jnp.dot(h2.astype(bf16), mW3_ref[...],
                  preferred_element_type=f32) + mb3_ref[...]
    c2 = _silu(jnp.dot(hc.astype(bf16), cW2_ref[...],
                       preferred_element_type=f32) + cb2_ref[...])
    cw = jnp.dot(c2, cW3c_ref[...], preferred_element_type=f32) + cb3_ref[...]
    logit = (jnp.dot(ha, aW2c_ref[...], preferred_element_type=f32)
             + ab2_ref[...])
    w = jnp.exp(logit)
    msg_ref[...] = msg * w

    blk8 = msg.shape[0] // 8
    vec = pc_ref[...] - pr_ref[...]
    sq = vec * vec
    d2 = sq + _shl(sq, 1) + _shl(sq, 2)
    inv = 1.0 / (jnp.sqrt(d2 + 1e-12) + 1e-8)
    lane = lax.broadcasted_iota(jnp.int32, (blk8, 128), 1)
    m0 = (lane & 15) == 0
    invm = jnp.where(m0, inv, 0.0)
    invb = _shr(invm, 1) + _shr(invm, 2) + _shr(invm, 3)
    dirp = _shr(vec, 1) * invb
    l8 = jnp.reshape(logit, (blk8, 8))
    c8 = jnp.reshape(cw, (blk8, 8))
    lp = jnp.dot(l8, sw_ref[...], preferred_element_type=f32)
    cp = jnp.dot(c8, sc_ref[...], preferred_element_type=f32)
    wp = jnp.where(m0, jnp.exp(lp), 0.0)
    tail_ref[...] = wp + cp * dirp


def _edge_mlps(xr, xc, pr8, pc8, rbf, Wr, Wc, Wb, b1, mW2, mb2r, mW3, mb3r,
               cW2, cb2r, cW3c, cb3r, aW2c, ab2r, Sw, Sc, E, D, NB, W1):
    BLK = 3200
    grid = (E // BLK,)
    espec = lambda wdt: pl.BlockSpec((BLK, wdt), lambda i: (i, 0))
    pspec = pl.BlockSpec((BLK // 8, 128), lambda i: (i, 0))
    full = lambda s: pl.BlockSpec(s, lambda i: (0, 0))
    return pl.pallas_call(
        functools.partial(_edge_body, D),
        grid=grid,
        in_specs=[
            espec(DW), espec(DW), pspec, pspec,
            pl.BlockSpec((NB, BLK), lambda i: (0, i)),
            full((D, W1)), full((D, W1)), full((NB, W1)), full((1, W1)),
            full((D, D)), full((1, D)), full((D, D)), full((1, D)),
            full((D, D)), full((1, D)), full((D, 1)), full((1, 1)),
            full((D // 2, 1)), full((1, 1)), full((8, 128)), full((8, 128)),
        ],
        out_specs=[espec(DW), pspec],
        out_shape=[jax.ShapeDtypeStruct((E, DW), jnp.float32),
                   jax.ShapeDtypeStruct((E // 8, 128), jnp.float32)],
        compiler_params=pltpu.CompilerParams(
            dimension_semantics=("parallel",)),
    )(xr, xc, pr8, pc8, rbf, Wr, Wc, Wb, b1, mW2, mb2r, mW3, mb3r,
      cW2, cb2r, cW3c, cb3r, aW2c, ab2r, Sw, Sc)


def _node_body(D, x_ref, pos_ref, m0_ref, m1_ref, t0_ref, t1_ref,
               uW1a_ref, uW1b_ref, ub1_ref, uW2_ref, ub2_ref,
               g_ref, b_ref, xo_ref, po_ref):
    f32 = jnp.float32
    num = m0_ref[...] + m1_ref[...]
    den = t0_ref[:, 0:1] + t1_ref[:, 0:1]
    ca = t0_ref[:, 1:4] + t1_ref[:, 1:4]
    m = num / (den + 1e-8)
    x = x_ref[...]
    h = _silu(jnp.dot(x, uW1a_ref[...], preferred_element_type=f32)
              + jnp.dot(m, uW1b_ref[...], preferred_element_type=f32)
              + ub1_ref[...])
    u = jnp.dot(h, uW2_ref[...], preferred_element_type=f32) + ub2_ref[...]
    pre = x + u
    mean = jnp.mean(pre, axis=1, keepdims=True)
    cent = pre - mean
    var = jnp.mean(cent * cent, axis=1, keepdims=True)
    xo_ref[...] = g_ref[...] * cent / jnp.sqrt(var + 1e-5) + b_ref[...]
    po_ref[...] = pos_ref[...] + ca


def _node_update(x, pos, m0, m1, t0, t1, uW1a, uW1b, ub1r, uW2, ub2r,
                 gr, br, n, D):
    BNK = 1000
    grid = (n // BNK,)
    full = lambda s: pl.BlockSpec(s, lambda i: (0, 0))
    return pl.pallas_call(
        functools.partial(_node_body, D),
        grid=grid,
        in_specs=[
            pl.BlockSpec((BNK, D), lambda i: (i, 0)),
            pl.BlockSpec((BNK, 3), lambda i: (i, 0)),
            pl.BlockSpec((BNK, DW), lambda i: (i, 0)),
            pl.BlockSpec((BNK, DW), lambda i: (i, 0)),
            pl.BlockSpec((BNK, PW), lambda i: (i, 0)),
            pl.BlockSpec((BNK, PW), lambda i: (i, 0)),
            full((D, D)), full((D, D)), full((1, D)), full((D, D)), full((1, D)),
            full((1, D)), full((1, D)),
        ],
        out_specs=[
            pl.BlockSpec((BNK, D), lambda i: (i, 0)),
            pl.BlockSpec((BNK, 3), lambda i: (i, 0)),
        ],
        out_shape=[
            jax.ShapeDtypeStruct((n, D), jnp.float32),
            jax.ShapeDtypeStruct((n, 3), jnp.float32),
        ],
        compiler_params=pltpu.CompilerParams(
            dimension_semantics=("parallel",)),
    )(x, pos, m0, m1, t0, t1, uW1a, uW1b, ub1r, uW2, ub2r, gr, br)


def kernel(x, pos, edge_index, rbf, mW1, mb1, mW2, mb2, mW3, mb3,
           aW1, ab1, aW2, ab2, cW1, cb1, cW2, cb2, cW3, cb3,
           uW1, ub1, uW2, ub2, gamma, beta):
    n, D = x.shape
    E = edge_index.shape[1]
    NB = rbf.shape[1]
    W1 = 2 * D + D // 2
    row = edge_index[0]
    col = edge_index[1]

    post = jnp.concatenate(
        [pos, jnp.zeros((n, PW - 3), jnp.float32)], axis=1)
    bf16 = jnp.bfloat16
    Wr = jnp.concatenate([mW1[:D], cW1[:D], aW1[:D]], axis=1).astype(bf16)
    Wc = jnp.concatenate([mW1[D:2 * D], cW1[D:2 * D], aW1[D:2 * D]],
                         axis=1).astype(bf16)
    Wb = jnp.concatenate([mW1[2 * D:], cW1[2 * D:], aW1[2 * D:]],
                         axis=1).astype(bf16)
    b1 = jnp.concatenate([mb1, cb1, ab1]).reshape(1, W1)

    ji = jnp.arange(8)[:, None]
    li = jnp.arange(128)[None, :]
    Sw = (li == 16 * ji).astype(jnp.float32)
    Sc = ((li >= 16 * ji + 1) & (li <= 16 * ji + 3)).astype(jnp.float32)

    xr, xc = _gather_x(x, row, col, E)
    pr, pc = _gather_pos(post, row, col, E)
    pr8 = jnp.reshape(pr, (E // 8, 128))
    pc8 = jnp.reshape(pc, (E // 8, 128))
    msg_pay, tail_pay = _edge_mlps(
        xr, xc, pr8, pc8, rbf.T, Wr, Wc, Wb, b1,
        mW2.astype(bf16), mb2.reshape(1, D), mW3.astype(bf16), mb3.reshape(1, D),
        cW2.astype(bf16), cb2.reshape(1, D), cW3.reshape(D, 1), cb3.reshape(1, 1),
        aW2.reshape(D // 2, 1), ab2.reshape(1, 1), Sw, Sc, E, D, NB, W1)
    accm = _scatter_msg(msg_pay, col, E)
    acct = _scatter_tail(jnp.reshape(tail_pay, (E, PW)), col, E)
    x_new, pos_new = _node_update(
        x, pos, accm[0, :n], accm[1, :n], acct[0, :n], acct[1, :n],
        uW1[:D], uW1[D:], ub1.reshape(1, D), uW2, ub2.reshape(1, D),
        gamma.reshape(1, D), beta.reshape(1, D), n, D)
    return (x_new, pos_new)


# speedup vs baseline: 1.1856x; 1.1149x over previous
import functools

import jax
import jax.numpy as jnp
from jax import lax
from jax.experimental import pallas as pl
from jax.experimental.pallas import tpu as pltpu
from jax.experimental.pallas import tpu_sc as plsc

NC = 2
NS = 16
NW = NC * NS
DW = 128
PW = 16
NP = 10240
CH = 200
CHP = 1000
CHS = 200
CHT = 1000


def _silu(v):
    return v * (0.5 * jnp.tanh(0.5 * v) + 0.5)


def _gather_body(epw, ch, t_hbm, row_hbm, col_hbm, outr_hbm, outc_hbm,
                 idxr, idxc, bufr, bufc, sem):
    wid = lax.axis_index("s") * NC + lax.axis_index("c")
    base0 = wid * epw

    def step(i, carry):
        base = base0 + i * ch
        pltpu.sync_copy(row_hbm.at[pl.ds(base, ch)], idxr)
        pltpu.sync_copy(col_hbm.at[pl.ds(base, ch)], idxc)
        a = pltpu.async_copy(t_hbm.at[idxr], bufr, sem)
        b = pltpu.async_copy(t_hbm.at[idxc], bufc, sem)
        a.wait()
        b.wait()
        pltpu.sync_copy(bufr, outr_hbm.at[pl.ds(base, ch)])
        pltpu.sync_copy(bufc, outc_hbm.at[pl.ds(base, ch)])
        return carry

    lax.fori_loop(0, epw // ch, step, 0)


def _gather_x(table, row, col, E):
    epw = E // NW
    return pl.kernel(
        functools.partial(_gather_body, epw, CH),
        out_type=[jax.ShapeDtypeStruct((E, DW), jnp.float32),
                  jax.ShapeDtypeStruct((E, DW), jnp.float32)],
        mesh=plsc.VectorSubcoreMesh(core_axis_name="c", subcore_axis_name="s"),
        scratch_types=[
            pltpu.VMEM((CH,), jnp.int32),
            pltpu.VMEM((CH,), jnp.int32),
            pltpu.VMEM((CH, DW), jnp.float32),
            pltpu.VMEM((CH, DW), jnp.float32),
            pltpu.SemaphoreType.DMA,
        ],
    )(table, row, col)


def _gather_pos(table, row, col, E):
    epw = E // NW
    return pl.kernel(
        functools.partial(_gather_body, epw, CHP),
        out_type=[jax.ShapeDtypeStruct((E, PW), jnp.float32),
                  jax.ShapeDtypeStruct((E, PW), jnp.float32)],
        mesh=plsc.VectorSubcoreMesh(core_axis_name="c", subcore_axis_name="s"),
        scratch_types=[
            pltpu.VMEM((CHP,), jnp.int32),
            pltpu.VMEM((CHP,), jnp.int32),
            pltpu.VMEM((CHP, PW), jnp.float32),
            pltpu.VMEM((CHP, PW), jnp.float32),
            pltpu.SemaphoreType.DMA,
        ],
        compiler_params=pltpu.CompilerParams(use_tc_tiling_on_sc=False),
    )(table, row, col)


def _scatter_body(epw, ch, w, payload_hbm, col_hbm, zeros_hbm, out_hbm,
                  idx, buf, acc):
    cid = lax.axis_index("c")
    sid = lax.axis_index("s")
    wid = sid * NC + cid
    rpt = NP // NS
    pltpu.sync_copy(zeros_hbm, acc.at[pl.ds(sid * rpt, rpt)])
    plsc.subcore_barrier()
    base0 = wid * epw

    def step(i, carry):
        base = base0 + i * ch
        pltpu.sync_copy(col_hbm.at[pl.ds(base, ch)], idx)
        pltpu.sync_copy(payload_hbm.at[pl.ds(base, ch)], buf)
        pltpu.sync_copy(buf, acc.at[idx], add=True)
        return carry

    lax.fori_loop(0, epw // ch, step, 0)
    plsc.subcore_barrier()
    pltpu.sync_copy(acc.at[pl.ds(sid * rpt, rpt)],
                    out_hbm.at[cid, pl.ds(sid * rpt, rpt)])


def _scatter_msg(payload, col, E):
    epw = E // NW
    zeros = jnp.zeros((NP // NS, DW), jnp.float32)
    return pl.kernel(
        functools.partial(_scatter_body, epw, CHS, DW),
        out_type=jax.ShapeDtypeStruct((NC, NP, DW), jnp.float32),
        mesh=plsc.VectorSubcoreMesh(core_axis_name="c", subcore_axis_name="s"),
        scratch_types=[
            pltpu.VMEM((CHS,), jnp.int32),
            pltpu.VMEM((CHS, DW), jnp.float32),
            pltpu.VMEM_SHARED((NP, DW), jnp.float32),
        ],
    )(payload, col, zeros)


def _scatter_tail(payload, col, E):
    epw = E // NW
    zeros = jnp.zeros((NP // NS, PW), jnp.float32)
    return pl.kernel(
        functools.partial(_scatter_body, epw, CHT, PW),
        out_type=jax.ShapeDtypeStruct((NC, NP, PW), jnp.float32),
        mesh=plsc.VectorSubcoreMesh(core_axis_name="c", subcore_axis_name="s"),
        scratch_types=[
            pltpu.VMEM((CHT,), jnp.int32),
            pltpu.VMEM((CHT, PW), jnp.float32),
            pltpu.VMEM_SHARED((NP, PW), jnp.float32),
        ],
        compiler_params=pltpu.CompilerParams(use_tc_tiling_on_sc=False),
    )(payload, col, zeros)


def _shl(x, k):
    z = jnp.zeros((x.shape[0], k), jnp.float32)
    return jnp.concatenate([x[:, k:], z], axis=1)


def _shr(x, k):
    z = jnp.zeros((x.shape[0], k), jnp.float32)
    return jnp.concatenate([z, x[:, :-k]], axis=1)


def _edge_body(D, xr_ref, xc_ref, pr_ref, pc_ref, rbf_ref,
               wr_ref, wc_ref, wb_ref, b1_ref,
               mW2_ref, mb2_ref, mW3_ref, mb3_ref,
               cW2_ref, cb2_ref, cW3c_ref, cb3_ref,
               aW2c_ref, ab2_ref, sw_ref, sc_ref, msg_ref, tail_ref):
    f32 = jnp.float32
    bf16 = jnp.bfloat16
    xr = xr_ref[...].astype(bf16)
    xc = xc_ref[...].astype(bf16)
    pre1 = (jnp.dot(xr, wr_ref[...], preferred_element_type=f32)
            + jnp.dot(xc, wc_ref[...], preferred_element_type=f32)
            + lax.dot_general(rbf_ref[...].astype(bf16), wb_ref[...],
                              (((0,), (0,)), ((), ())),
                              preferred_element_type=f32)
            + b1_ref[...])
    hm = _silu(pre1[:, :D])
    hc = _silu(pre1[:, D:2 * D])
    ha = _silu(pre1[:, 2 * D:])
    h2 = _silu(jnp.dot(hm.astype(bf16), mW2_ref[...],
                       preferred_element_type=f32) + mb2_ref[...])
    msg = jnp.dot(h2.astype(bf16), mW3_ref[...],
                  preferred_element_type=f32) + mb3_ref[...]
    c2 = _silu(jnp.dot(hc.astype(bf16), cW2_ref[...],
                       preferred_element_type=f32) + cb2_ref[...])
    cw = jnp.dot(c2, cW3c_ref[...], preferred_element_type=f32) + cb3_ref[...]
    logit = (jnp.dot(ha, aW2c_ref[...], preferred_element_type=f32)
             + ab2_ref[...])
    w = jnp.exp(logit)
    msg_ref[...] = msg * w

    blk8 = msg.shape[0] // 8
    vec = pc_ref[...] - pr_ref[...]
    sq = vec * vec
    d2 = sq + _shl(sq, 1) + _shl(sq, 2)
    inv = 1.0 / (jnp.sqrt(d2 + 1e-12) + 1e-8)
    lane = lax.broadcasted_iota(jnp.int32, (blk8, 128), 1)
    m0 = (lane & 15) == 0
    invm = jnp.where(m0, inv, 0.0)
    invb = _shr(invm, 1) + _shr(invm, 2) + _shr(invm, 3)
    dirp = _shr(vec, 1) * invb
    l8 = jnp.reshape(logit, (blk8, 8))
    c8 = jnp.reshape(cw, (blk8, 8))
    lp = jnp.dot(l8, sw_ref[...], preferred_element_type=f32)
    cp = jnp.dot(c8, sc_ref[...], preferred_element_type=f32)
    wp = jnp.where(m0, jnp.exp(lp), 0.0)
    tail_ref[...] = wp + cp * dirp


def _edge_mlps(xr, xc, pr8, pc8, rbf, Wr, Wc, Wb, b1, mW2, mb2r, mW3, mb3r,
               cW2, cb2r, cW3c, cb3r, aW2c, ab2r, Sw, Sc, E, D, NB, W1, off):
    BLK = 3200
    grid = (E // BLK,)
    espec = lambda wdt: pl.BlockSpec((BLK, wdt), lambda i: (i, 0))
    pspec = pl.BlockSpec((BLK // 8, 128), lambda i: (i, 0))
    full = lambda s: pl.BlockSpec(s, lambda i: (0, 0))
    return pl.pallas_call(
        functools.partial(_edge_body, D),
        grid=grid,
        in_specs=[
            espec(DW), espec(DW), pspec, pspec,
            pl.BlockSpec((NB, BLK), lambda i: (0, i + off)),
            full((D, W1)), full((D, W1)), full((NB, W1)), full((1, W1)),
            full((D, D)), full((1, D)), full((D, D)), full((1, D)),
            full((D, D)), full((1, D)), full((D, 1)), full((1, 1)),
            full((D // 2, 1)), full((1, 1)), full((8, 128)), full((8, 128)),
        ],
        out_specs=[espec(DW), pspec],
        out_shape=[jax.ShapeDtypeStruct((E, DW), jnp.float32),
                   jax.ShapeDtypeStruct((E // 8, 128), jnp.float32)],
        compiler_params=pltpu.CompilerParams(
            dimension_semantics=("parallel",)),
    )(xr, xc, pr8, pc8, rbf, Wr, Wc, Wb, b1, mW2, mb2r, mW3, mb3r,
      cW2, cb2r, cW3c, cb3r, aW2c, ab2r, Sw, Sc)


def _node_body(D, x_ref, pos_ref, m0_ref, m1_ref, m2_ref, m3_ref,
               t0_ref, t1_ref, t2_ref, t3_ref,
               uW1a_ref, uW1b_ref, ub1_ref, uW2_ref, ub2_ref,
               g_ref, b_ref, xo_ref, po_ref):
    f32 = jnp.float32
    num = (m0_ref[...] + m1_ref[...]) + (m2_ref[...] + m3_ref[...])
    tsum = (t0_ref[...] + t1_ref[...]) + (t2_ref[...] + t3_ref[...])
    den = tsum[:, 0:1]
    ca = tsum[:, 1:4]
    m = num / (den + 1e-8)
    x = x_ref[...]
    h = _silu(jnp.dot(x, uW1a_ref[...], preferred_element_type=f32)
              + jnp.dot(m, uW1b_ref[...], preferred_element_type=f32)
              + ub1_ref[...])
    u = jnp.dot(h, uW2_ref[...], preferred_element_type=f32) + ub2_ref[...]
    pre = x + u
    mean = jnp.mean(pre, axis=1, keepdims=True)
    cent = pre - mean
    var = jnp.mean(cent * cent, axis=1, keepdims=True)
    xo_ref[...] = g_ref[...] * cent / jnp.sqrt(var + 1e-5) + b_ref[...]
    po_ref[...] = pos_ref[...] + ca


def _node_update(x, pos, ms, ts, uW1a, uW1b, ub1r, uW2, ub2r,
                 gr, br, n, D):
    BNK = 1000
    grid = (n // BNK,)
    full = lambda s: pl.BlockSpec(s, lambda i: (0, 0))
    return pl.pallas_call(
        functools.partial(_node_body, D),
        grid=grid,
        in_specs=[
            pl.BlockSpec((BNK, D), lambda i: (i, 0)),
            pl.BlockSpec((BNK, 3), lambda i: (i, 0)),
        ] + [pl.BlockSpec((BNK, DW), lambda i: (i, 0))] * 4
          + [pl.BlockSpec((BNK, PW), lambda i: (i, 0))] * 4
          + [
            full((D, D)), full((D, D)), full((1, D)), full((D, D)), full((1, D)),
            full((1, D)), full((1, D)),
        ],
        out_specs=[
            pl.BlockSpec((BNK, D), lambda i: (i, 0)),
            pl.BlockSpec((BNK, 3), lambda i: (i, 0)),
        ],
        out_shape=[
            jax.ShapeDtypeStruct((n, D), jnp.float32),
            jax.ShapeDtypeStruct((n, 3), jnp.float32),
        ],
        compiler_params=pltpu.CompilerParams(
            dimension_semantics=("parallel",)),
    )(x, pos, *ms, *ts, uW1a, uW1b, ub1r, uW2, ub2r, gr, br)


def kernel(x, pos, edge_index, rbf, mW1, mb1, mW2, mb2, mW3, mb3,
           aW1, ab1, aW2, ab2, cW1, cb1, cW2, cb2, cW3, cb3,
           uW1, ub1, uW2, ub2, gamma, beta):
    n, D = x.shape
    E = edge_index.shape[1]
    NB = rbf.shape[1]
    W1 = 2 * D + D // 2
    row = edge_index[0]
    col = edge_index[1]

    post = jnp.concatenate(
        [pos, jnp.zeros((n, PW - 3), jnp.float32)], axis=1)
    bf16 = jnp.bfloat16
    Wr = jnp.concatenate([mW1[:D], cW1[:D], aW1[:D]], axis=1).astype(bf16)
    Wc = jnp.concatenate([mW1[D:2 * D], cW1[D:2 * D], aW1[D:2 * D]],
                         axis=1).astype(bf16)
    Wb = jnp.concatenate([mW1[2 * D:], cW1[2 * D:], aW1[2 * D:]],
                         axis=1).astype(bf16)
    b1 = jnp.concatenate([mb1, cb1, ab1]).reshape(1, W1)

    ji = jnp.arange(8)[:, None]
    li = jnp.arange(128)[None, :]
    Sw = (li == 16 * ji).astype(jnp.float32)
    Sc = ((li >= 16 * ji + 1) & (li <= 16 * ji + 3)).astype(jnp.float32)

    E2 = E // 2
    rbft = rbf.T
    wts = (mW2.astype(bf16), mb2.reshape(1, D), mW3.astype(bf16),
           mb3.reshape(1, D), cW2.astype(bf16), cb2.reshape(1, D),
           cW3.reshape(D, 1), cb3.reshape(1, 1),
           aW2.reshape(D // 2, 1), ab2.reshape(1, 1))
    ms, ts = [], []
    for h in (0, 1):
        rh = lax.dynamic_slice_in_dim(row, h * E2, E2)
        ch = lax.dynamic_slice_in_dim(col, h * E2, E2)
        xr, xc = _gather_x(x, rh, ch, E2)
        pr, pc = _gather_pos(post, rh, ch, E2)
        pr8 = jnp.reshape(pr, (E2 // 8, 128))
        pc8 = jnp.reshape(pc, (E2 // 8, 128))
        msg_pay, tail_pay = _edge_mlps(
            xr, xc, pr8, pc8, rbft, Wr, Wc, Wb, b1, *wts, Sw, Sc,
            E2, D, NB, W1, h * (E2 // 3200))
        accm = _scatter_msg(msg_pay, ch, E2)
        acct = _scatter_tail(jnp.reshape(tail_pay, (E2, PW)), ch, E2)
        ms += [accm[0, :n], accm[1, :n]]
        ts += [acct[0, :n], acct[1, :n]]
    x_new, pos_new = _node_update(
        x, pos, ms, ts,
        uW1[:D], uW1[D:], ub1.reshape(1, D), uW2, ub2.reshape(1, D),
        gamma.reshape(1, D), beta.reshape(1, D), n, D)
    return (x_new, pos_new)
